# baseline (speedup 1.0000x reference)
import functools

import jax
import jax.numpy as jnp
from jax import lax
from jax.experimental import pallas as pl
from jax.experimental.pallas import tpu as pltpu
from jax.experimental.pallas import tpu_sc as plsc

N = 10000
D = 128
HD = D // 2
E = 320000
NW = 32
B = 64
CH = 158
EW = B * CH
EPAD = EW * NW
NR = 10112
RPT = NR // 16
TB = 128


def _rowblock(i):
    return (i, 0)


def _pack_rows(xn):
    lo = lax.bitcast_convert_type(xn[:, :HD].astype(jnp.bfloat16), jnp.uint16)
    hi = lax.bitcast_convert_type(xn[:, HD:].astype(jnp.bfloat16), jnp.uint16)
    pk = (hi.astype(jnp.uint32) << 16) | lo.astype(jnp.uint32)
    return pk.astype(jnp.int32)


def _tc_invnorm(x):
    def body(x_ref, pk_ref, nr_ref):
        xb = x_ref[...]
        s = jnp.sum(xb * xb, axis=1, keepdims=True)
        nrm = jnp.sqrt(s)
        xn = xb / jnp.maximum(nrm, 1e-12)
        pk_ref[...] = _pack_rows(xn)
        nr_ref[...] = nrm

    return pl.pallas_call(
        body,
        grid=(NR // TB,),
        in_specs=[pl.BlockSpec((TB, D), _rowblock)],
        out_specs=[pl.BlockSpec((TB, HD), _rowblock),
                   pl.BlockSpec((TB, 1), _rowblock)],
        out_shape=[jax.ShapeDtypeStruct((NR, HD), jnp.int32),
                   jax.ShapeDtypeStruct((NR, 1), jnp.float32)],
    )(x)


def _tc_mid(p0, p1, d0, d1):
    def body(p0r, p1r, d0r, d1r, pk_ref, nr_ref):
        s = p0r[...] + p1r[...]
        den = d0r[...] + d1r[...] + 1e-12
        h = jnp.maximum(s / den, 0.0)
        n2 = jnp.sum(h * h, axis=1, keepdims=True)
        nrm = jnp.sqrt(n2)
        hn = h / jnp.maximum(nrm, 1e-12)
        pk_ref[...] = _pack_rows(hn)
        nr_ref[...] = nrm

    return pl.pallas_call(
        body,
        grid=(NR // TB,),
        in_specs=[pl.BlockSpec((TB, D), _rowblock), pl.BlockSpec((TB, D), _rowblock),
                  pl.BlockSpec((TB, 1), _rowblock), pl.BlockSpec((TB, 1), _rowblock)],
        out_specs=[pl.BlockSpec((TB, HD), _rowblock),
                   pl.BlockSpec((TB, 1), _rowblock)],
        out_shape=[jax.ShapeDtypeStruct((NR, HD), jnp.int32),
                   jax.ShapeDtypeStruct((NR, 1), jnp.float32)],
    )(p0, p1, d0, d1)


def _tc_final(p0, p1, d0, d1):
    def body(p0r, p1r, d0r, d1r, hr):
        s = p0r[...] + p1r[...]
        den = d0r[...] + d1r[...] + 1e-12
        hr[...] = s / den

    return pl.pallas_call(
        body,
        grid=((N + TB - 1) // TB,),
        in_specs=[pl.BlockSpec((TB, D), _rowblock), pl.BlockSpec((TB, D), _rowblock),
                  pl.BlockSpec((TB, 1), _rowblock), pl.BlockSpec((TB, 1), _rowblock)],
        out_specs=pl.BlockSpec((TB, D), _rowblock),
        out_shape=jax.ShapeDtypeStruct((N, D), jnp.float32),
    )(p0, p1, d0, d1)


def _unpack_lo(v):
    return plsc.bitcast(lax.shift_left(v, 16), jnp.float32)


def _unpack_hi(v):
    return plsc.bitcast(v & jnp.int32(-65536), jnp.float32)


def _sc_body(x_hbm, nrm_hbm, beta_hbm, eidx_hbm,
             out_hbm, den_hbm,
             eidx0, eidx1, sdidx0, sdidx1,
             combbuf0, combbuf1, sctbuf0, sctbuf1,
             wbuf0, wbuf1, wnbuf0, wnbuf1, nrm_v, beta_v, zden,
             out_sh, den_sh,
             gs0, gs1, ss0, ss1, dd0, dd1, is_):
    cid = lax.axis_index("c")
    tid = lax.axis_index("s")
    wid = cid * 16 + tid

    EIDX = (eidx0, eidx1)
    SDIDX = (sdidx0, sdidx1)
    COMB = (combbuf0, combbuf1)
    SCT = (sctbuf0, sctbuf1)
    WB = (wbuf0, wbuf1)
    WN = (wnbuf0, wnbuf1)
    GS = (gs0, gs1)
    SS = (ss0, ss1)
    DDS = (dd0, dd1)

    z16 = jnp.zeros((16,), jnp.float32)

    def zr(i, _):
        sctbuf0[i // 8, pl.ds((i % 8) * 16, 16)] = z16
        return 0
    lax.fori_loop(0, B * 8, zr, 0)

    def zd(i, _):
        zden[pl.ds(i * 16, 16)] = z16
        return 0
    lax.fori_loop(0, 40, zd, 0)

    base_r = tid * RPT
    for k in range(RPT // B):
        pltpu.sync_copy(sctbuf0, out_sh.at[pl.ds(base_r + k * B, B)])
    rem = RPT - (RPT // B) * B
    if rem:
        pltpu.sync_copy(sctbuf0.at[pl.ds(0, rem)],
                        out_sh.at[pl.ds(base_r + RPT - rem, rem)])
    pltpu.sync_copy(zden.at[pl.ds(0, RPT)], den_sh.at[pl.ds(base_r, RPT)])

    pltpu.sync_copy(nrm_hbm, nrm_v)
    pltpu.sync_copy(beta_hbm, beta_v)
    betav = beta_v[...]
    plsc.subcore_barrier()

    iota16 = lax.iota(jnp.int32, 16)

    for s in range(2):
        i0 = (wid * CH + s) * 2 * B
        pltpu.sync_copy(eidx_hbm.at[pl.ds(i0, 2 * B)], EIDX[s])
    for s in range(2):
        pltpu.async_copy(x_hbm.at[EIDX[s]], COMB[s], GS[s])

    def one_chunk(c, s):
        @pl.when(c >= 2)
        def _():
            pltpu.make_async_copy(SCT[s], out_sh.at[SDIDX[s]], SS[s]).wait()
            pltpu.make_async_copy(WB[s], den_sh.at[SDIDX[s]], DDS[s]).wait()

        pltpu.make_async_copy(x_hbm.at[EIDX[s]], COMB[s], GS[s]).wait()

        base = wid * EW + c * B
        nsg = []
        for g in range(B // 16):
            sig = EIDX[s][pl.ds(g * 16, 16)]
            dig = EIDX[s][pl.ds(B + g * 16, 16)]
            SDIDX[s][pl.ds(g * 16, 16)] = dig
            nsg.append(plsc.load_gather(nrm_v, [sig]))

        @pl.when(c + 2 < CH)
        def _():
            i2 = (wid * CH + c + 2) * 2 * B
            pltpu.async_copy(eidx_hbm.at[pl.ds(i2, 2 * B)], EIDX[s], is_)

        def edot(b, _):
            acc = z16
            for j in range(HD // 16):
                vs = COMB[s][b, pl.ds(j * 16, 16)]
                vd = COMB[s][B + b, pl.ds(j * 16, 16)]
                acc = acc + _unpack_lo(vs) * _unpack_lo(vd)
                acc = acc + _unpack_hi(vs) * _unpack_hi(vd)
            SCT[s][b, pl.ds(0, 16)] = plsc.cumsum(acc)
            return 0
        lax.fori_loop(0, B, edot, 0, unroll=8)

        fifteen = jnp.full((16,), 15, jnp.int32)
        for g in range(B // 16):
            rows = g * 16 + iota16
            dots = plsc.load_gather(SCT[s], [rows, fifteen])
            wv = jnp.exp(dots * betav)
            gidx = base + g * 16 + iota16
            wv = jnp.where(gidx < E, wv, 0.0)
            WB[s][pl.ds(g * 16, 16)] = wv
            WN[s][pl.ds(g * 16, 16)] = wv * nsg[g]

        def escale(b, _):
            bsp = jnp.full((16,), b, jnp.int32)
            wsp = plsc.load_gather(WN[s], [bsp])
            for j in range(HD // 16):
                v = COMB[s][b, pl.ds(j * 16, 16)]
                SCT[s][b, pl.ds(j * 16, 16)] = _unpack_lo(v) * wsp
                SCT[s][b, pl.ds(HD + j * 16, 16)] = _unpack_hi(v) * wsp
            return 0
        lax.fori_loop(0, B, escale, 0, unroll=8)

        pltpu.async_copy(SCT[s], out_sh.at[SDIDX[s]], SS[s], add=True)
        pltpu.async_copy(WB[s], den_sh.at[SDIDX[s]], DDS[s], add=True)

        @pl.when(c + 2 < CH)
        def _():
            i2 = (wid * CH + c + 2) * 2 * B
            pltpu.make_async_copy(eidx_hbm.at[pl.ds(i2, 2 * B)], EIDX[s],
                                  is_).wait()
            pltpu.async_copy(x_hbm.at[EIDX[s]], COMB[s], GS[s])

    def outer(cc, _):
        one_chunk(2 * cc, 0)
        one_chunk(2 * cc + 1, 1)
        return 0
    lax.fori_loop(0, CH // 2, outer, 0)

    for s in range(2):
        pltpu.make_async_copy(SCT[s], out_sh.at[SDIDX[s]], SS[s]).wait()
        pltpu.make_async_copy(WB[s], den_sh.at[SDIDX[s]], DDS[s]).wait()

    plsc.subcore_barrier()
    pltpu.sync_copy(out_sh.at[pl.ds(base_r, RPT)],
                    out_hbm.at[cid, pl.ds(base_r, RPT)])
    pltpu.sync_copy(den_sh.at[pl.ds(base_r, RPT)], zden.at[pl.ds(0, RPT)])
    pltpu.sync_copy(zden.at[pl.ds(0, RPT)],
                    den_hbm.at[pl.ds(cid * NR + base_r, RPT)])


def _sc_layer(xpk, nrm, betav, eidx):
    f32 = jnp.float32
    i32 = jnp.int32
    mesh = plsc.VectorSubcoreMesh(core_axis_name="c", subcore_axis_name="s")
    kern = pl.kernel(
        _sc_body,
        out_type=[jax.ShapeDtypeStruct((2, NR, D), f32),
                  jax.ShapeDtypeStruct((2 * NR,), f32)],
        mesh=mesh,
        scratch_types=[
            pltpu.VMEM((2 * B,), i32),
            pltpu.VMEM((2 * B,), i32),
            pltpu.VMEM((B,), i32),
            pltpu.VMEM((B,), i32),
            pltpu.VMEM((2 * B, HD), i32),
            pltpu.VMEM((2 * B, HD), i32),
            pltpu.VMEM((B, D), f32),
            pltpu.VMEM((B, D), f32),
            pltpu.VMEM((B,), f32),
            pltpu.VMEM((B,), f32),
            pltpu.VMEM((B,), f32),
            pltpu.VMEM((B,), f32),
            pltpu.VMEM((NR,), f32),
            pltpu.VMEM((16,), f32),
            pltpu.VMEM((640,), f32),
            pltpu.VMEM_SHARED((NR, D), f32),
            pltpu.VMEM_SHARED((NR,), f32),
        ] + [pltpu.SemaphoreType.DMA] * 7,
        compiler_params=pltpu.CompilerParams(needs_layout_passes=False,
                                             use_tc_tiling_on_sc=False),
    )
    return kern(xpk, nrm, betav, eidx)


def kernel(x, edge_index, beta1, beta2):
    src = edge_index[0].astype(jnp.int32)
    dst = edge_index[1].astype(jnp.int32)
    src = jnp.pad(src, (0, EPAD - E))
    dst = jnp.pad(dst, (0, EPAD - E))
    eidx = jnp.stack([src.reshape(NW * CH, B), dst.reshape(NW * CH, B)],
                     axis=1).reshape(2 * EPAD)
    xp = jnp.pad(x, ((0, NR - N), (0, 0)))

    xpk, nrm1 = _tc_invnorm(xp)
    b1 = jnp.full((16,), beta1, jnp.float32)
    p, d = _sc_layer(xpk, nrm1.reshape(NR), b1, eidx)
    d = d.reshape(2, NR)
    hpk, nrm2 = _tc_mid(p[0], p[1], d[0].reshape(NR, 1), d[1].reshape(NR, 1))
    b2 = jnp.full((16,), beta2, jnp.float32)
    p2, d2 = _sc_layer(hpk, nrm2.reshape(NR), b2, eidx)
    d2 = d2.reshape(2, NR)
    return _tc_final(p2[0], p2[1], d2[0].reshape(NR, 1), d2[1].reshape(NR, 1))


# speedup vs baseline: 1.0064x; 1.0064x over previous
import functools

import jax
import jax.numpy as jnp
from jax import lax
from jax.experimental import pallas as pl
from jax.experimental.pallas import tpu as pltpu
from jax.experimental.pallas import tpu_sc as plsc

N = 10000
D = 128
HD = D // 2
E = 320000
NW = 32
B = 64
CH = 158
EW = B * CH
EPAD = EW * NW
NR = 10112
RPT = NR // 16
TB = 128


def _rowblock(i):
    return (i, 0)


def _pack_rows(xn):
    lo = lax.bitcast_convert_type(xn[:, :HD].astype(jnp.bfloat16), jnp.uint16)
    hi = lax.bitcast_convert_type(xn[:, HD:].astype(jnp.bfloat16), jnp.uint16)
    pk = (hi.astype(jnp.uint32) << 16) | lo.astype(jnp.uint32)
    return pk.astype(jnp.int32)


def _tc_invnorm(x):
    def body(x_ref, pk_ref, nr_ref):
        xb = x_ref[...]
        s = jnp.sum(xb * xb, axis=1, keepdims=True)
        nrm = jnp.sqrt(s)
        xn = xb / jnp.maximum(nrm, 1e-12)
        pk_ref[...] = _pack_rows(xn)
        nr_ref[...] = nrm

    return pl.pallas_call(
        body,
        grid=(NR // TB,),
        in_specs=[pl.BlockSpec((TB, D), _rowblock)],
        out_specs=[pl.BlockSpec((TB, HD), _rowblock),
                   pl.BlockSpec((TB, 1), _rowblock)],
        out_shape=[jax.ShapeDtypeStruct((NR, HD), jnp.int32),
                   jax.ShapeDtypeStruct((NR, 1), jnp.float32)],
    )(x)


def _tc_mid(p0, p1, d0, d1):
    def body(p0r, p1r, d0r, d1r, pk_ref, nr_ref):
        s = p0r[...] + p1r[...]
        den = d0r[...] + d1r[...] + 1e-12
        h = jnp.maximum(s / den, 0.0)
        n2 = jnp.sum(h * h, axis=1, keepdims=True)
        nrm = jnp.sqrt(n2)
        hn = h / jnp.maximum(nrm, 1e-12)
        pk_ref[...] = _pack_rows(hn)
        nr_ref[...] = nrm

    return pl.pallas_call(
        body,
        grid=(NR // TB,),
        in_specs=[pl.BlockSpec((TB, D), _rowblock), pl.BlockSpec((TB, D), _rowblock),
                  pl.BlockSpec((TB, 1), _rowblock), pl.BlockSpec((TB, 1), _rowblock)],
        out_specs=[pl.BlockSpec((TB, HD), _rowblock),
                   pl.BlockSpec((TB, 1), _rowblock)],
        out_shape=[jax.ShapeDtypeStruct((NR, HD), jnp.int32),
                   jax.ShapeDtypeStruct((NR, 1), jnp.float32)],
    )(p0, p1, d0, d1)


def _tc_final(p0, p1, d0, d1):
    def body(p0r, p1r, d0r, d1r, hr):
        s = p0r[...] + p1r[...]
        den = d0r[...] + d1r[...] + 1e-12
        hr[...] = s / den

    return pl.pallas_call(
        body,
        grid=(NR // TB,),
        in_specs=[pl.BlockSpec((TB, D), _rowblock), pl.BlockSpec((TB, D), _rowblock),
                  pl.BlockSpec((TB, 1), _rowblock), pl.BlockSpec((TB, 1), _rowblock)],
        out_specs=pl.BlockSpec((TB, D), _rowblock),
        out_shape=jax.ShapeDtypeStruct((NR, D), jnp.float32),
    )(p0, p1, d0, d1)


def _unpack_lo(v):
    return plsc.bitcast(lax.shift_left(v, 16), jnp.float32)


def _unpack_hi(v):
    return plsc.bitcast(v & jnp.int32(-65536), jnp.float32)


def _sc_body(x_hbm, nrm_hbm, beta_hbm, eidx_hbm,
             out_hbm, den_hbm,
             eidx0, eidx1, sdidx0, sdidx1,
             combbuf0, combbuf1, sctbuf0, sctbuf1,
             wbuf0, wbuf1, wnbuf0, wnbuf1, nrm_v, beta_v, zden,
             out_sh, den_sh,
             gs0, gs1, ss0, ss1, dd0, dd1, is_):
    cid = lax.axis_index("c")
    tid = lax.axis_index("s")
    wid = cid * 16 + tid

    EIDX = (eidx0, eidx1)
    SDIDX = (sdidx0, sdidx1)
    COMB = (combbuf0, combbuf1)
    SCT = (sctbuf0, sctbuf1)
    WB = (wbuf0, wbuf1)
    WN = (wnbuf0, wnbuf1)
    GS = (gs0, gs1)
    SS = (ss0, ss1)
    DDS = (dd0, dd1)

    z16 = jnp.zeros((16,), jnp.float32)

    def zr(i, _):
        sctbuf0[i // 8, pl.ds((i % 8) * 16, 16)] = z16
        return 0
    lax.fori_loop(0, B * 8, zr, 0)

    def zd(i, _):
        zden[pl.ds(i * 16, 16)] = z16
        return 0
    lax.fori_loop(0, 40, zd, 0)

    base_r = tid * RPT
    for k in range(RPT // B):
        pltpu.sync_copy(sctbuf0, out_sh.at[pl.ds(base_r + k * B, B)])
    rem = RPT - (RPT // B) * B
    if rem:
        pltpu.sync_copy(sctbuf0.at[pl.ds(0, rem)],
                        out_sh.at[pl.ds(base_r + RPT - rem, rem)])
    pltpu.sync_copy(zden.at[pl.ds(0, RPT)], den_sh.at[pl.ds(base_r, RPT)])

    pltpu.sync_copy(nrm_hbm, nrm_v)
    pltpu.sync_copy(beta_hbm, beta_v)
    betav = beta_v[...]
    plsc.subcore_barrier()

    iota16 = lax.iota(jnp.int32, 16)

    for s in range(2):
        i0 = (wid * CH + s) * 2 * B
        pltpu.sync_copy(eidx_hbm.at[pl.ds(i0, 2 * B)], EIDX[s])
    for s in range(2):
        pltpu.async_copy(x_hbm.at[EIDX[s]], COMB[s], GS[s])

    def one_chunk(c, s):
        @pl.when(c >= 2)
        def _():
            pltpu.make_async_copy(SCT[s], out_sh.at[SDIDX[s]], SS[s]).wait()
            pltpu.make_async_copy(WB[s], den_sh.at[SDIDX[s]], DDS[s]).wait()

        pltpu.make_async_copy(x_hbm.at[EIDX[s]], COMB[s], GS[s]).wait()

        base = wid * EW + c * B
        nsg = []
        for g in range(B // 16):
            sig = EIDX[s][pl.ds(g * 16, 16)]
            dig = EIDX[s][pl.ds(B + g * 16, 16)]
            SDIDX[s][pl.ds(g * 16, 16)] = dig
            nsg.append(plsc.load_gather(nrm_v, [sig]))

        @pl.when(c + 2 < CH)
        def _():
            i2 = (wid * CH + c + 2) * 2 * B
            pltpu.async_copy(eidx_hbm.at[pl.ds(i2, 2 * B)], EIDX[s], is_)

        def edot(b, _):
            acc = z16
            for j in range(HD // 16):
                vs = COMB[s][b, pl.ds(j * 16, 16)]
                vd = COMB[s][B + b, pl.ds(j * 16, 16)]
                acc = acc + _unpack_lo(vs) * _unpack_lo(vd)
                acc = acc + _unpack_hi(vs) * _unpack_hi(vd)
            SCT[s][b, pl.ds(0, 16)] = plsc.cumsum(acc)
            return 0
        lax.fori_loop(0, B, edot, 0, unroll=8)

        fifteen = jnp.full((16,), 15, jnp.int32)
        for g in range(B // 16):
            rows = g * 16 + iota16
            dots = plsc.load_gather(SCT[s], [rows, fifteen])
            wv = jnp.exp(dots * betav)
            gidx = base + g * 16 + iota16
            wv = jnp.where(gidx < E, wv, 0.0)
            WB[s][pl.ds(g * 16, 16)] = wv
            WN[s][pl.ds(g * 16, 16)] = wv * nsg[g]

        def escale(b, _):
            bsp = jnp.full((16,), b, jnp.int32)
            wsp = plsc.load_gather(WN[s], [bsp])
            for j in range(HD // 16):
                v = COMB[s][b, pl.ds(j * 16, 16)]
                SCT[s][b, pl.ds(j * 16, 16)] = _unpack_lo(v) * wsp
                SCT[s][b, pl.ds(HD + j * 16, 16)] = _unpack_hi(v) * wsp
            return 0
        lax.fori_loop(0, B, escale, 0, unroll=8)

        pltpu.async_copy(SCT[s], out_sh.at[SDIDX[s]], SS[s], add=True)
        pltpu.async_copy(WB[s], den_sh.at[SDIDX[s]], DDS[s], add=True)

        @pl.when(c + 2 < CH)
        def _():
            i2 = (wid * CH + c + 2) * 2 * B
            pltpu.make_async_copy(eidx_hbm.at[pl.ds(i2, 2 * B)], EIDX[s],
                                  is_).wait()
            pltpu.async_copy(x_hbm.at[EIDX[s]], COMB[s], GS[s])

    def outer(cc, _):
        one_chunk(2 * cc, 0)
        one_chunk(2 * cc + 1, 1)
        return 0
    lax.fori_loop(0, CH // 2, outer, 0)

    for s in range(2):
        pltpu.make_async_copy(SCT[s], out_sh.at[SDIDX[s]], SS[s]).wait()
        pltpu.make_async_copy(WB[s], den_sh.at[SDIDX[s]], DDS[s]).wait()

    plsc.subcore_barrier()
    pltpu.sync_copy(out_sh.at[pl.ds(base_r, RPT)],
                    out_hbm.at[cid, pl.ds(base_r, RPT)])
    pltpu.sync_copy(den_sh.at[pl.ds(base_r, RPT)], zden.at[pl.ds(0, RPT)])
    pltpu.sync_copy(zden.at[pl.ds(0, RPT)],
                    den_hbm.at[pl.ds(cid * NR + base_r, RPT)])


def _sc_layer(xpk, nrm, betav, eidx):
    f32 = jnp.float32
    i32 = jnp.int32
    mesh = plsc.VectorSubcoreMesh(core_axis_name="c", subcore_axis_name="s")
    kern = pl.kernel(
        _sc_body,
        out_type=[jax.ShapeDtypeStruct((2, NR, D), f32),
                  jax.ShapeDtypeStruct((2 * NR,), f32)],
        mesh=mesh,
        scratch_types=[
            pltpu.VMEM((2 * B,), i32),
            pltpu.VMEM((2 * B,), i32),
            pltpu.VMEM((B,), i32),
            pltpu.VMEM((B,), i32),
            pltpu.VMEM((2 * B, HD), i32),
            pltpu.VMEM((2 * B, HD), i32),
            pltpu.VMEM((B, D), f32),
            pltpu.VMEM((B, D), f32),
            pltpu.VMEM((B,), f32),
            pltpu.VMEM((B,), f32),
            pltpu.VMEM((B,), f32),
            pltpu.VMEM((B,), f32),
            pltpu.VMEM((NR,), f32),
            pltpu.VMEM((16,), f32),
            pltpu.VMEM((640,), f32),
            pltpu.VMEM_SHARED((NR, D), f32),
            pltpu.VMEM_SHARED((NR,), f32),
        ] + [pltpu.SemaphoreType.DMA] * 7,
        compiler_params=pltpu.CompilerParams(needs_layout_passes=False,
                                             use_tc_tiling_on_sc=False),
    )
    return kern(xpk, nrm, betav, eidx)


def kernel(x, edge_index, beta1, beta2):
    src = edge_index[0].astype(jnp.int32)
    dst = edge_index[1].astype(jnp.int32)
    src = jnp.pad(src, (0, EPAD - E))
    dst = jnp.pad(dst, (0, EPAD - E))
    eidx = jnp.stack([src.reshape(NW * CH, B), dst.reshape(NW * CH, B)],
                     axis=1).reshape(2 * EPAD)
    xp = jnp.pad(x, ((0, NR - N), (0, 0)))

    xpk, nrm1 = _tc_invnorm(xp)
    b1 = jnp.full((16,), beta1, jnp.float32)
    p, d = _sc_layer(xpk, nrm1.reshape(NR), b1, eidx)
    d = d.reshape(2, NR)
    hpk, nrm2 = _tc_mid(p[0], p[1], d[0].reshape(NR, 1), d[1].reshape(NR, 1))
    b2 = jnp.full((16,), beta2, jnp.float32)
    p2, d2 = _sc_layer(hpk, nrm2.reshape(NR), b2, eidx)
    d2 = d2.reshape(2, NR)
    out = _tc_final(p2[0], p2[1], d2[0].reshape(NR, 1), d2[1].reshape(NR, 1))
    return out[:N]
